# Initial kernel scaffold; baseline (speedup 1.0000x reference)
import functools

import jax
import jax.numpy as jnp
from jax import lax
from jax.experimental import pallas as pl
from jax.experimental.pallas import tpu as pltpu
from jax.experimental.pallas import tpu_sc as plsc

VAR_IDX = 3
D = 32
NC = 2
NS = 16
NW = NC * NS
SUB = 128
SUBS_PER_CHUNK = 8
CHUNK = SUB * SUBS_PER_CHUNK


def _make_gather(B):
    b_per_w = B // NW
    n_sub_rows = b_per_w // SUB
    n_chunks = b_per_w // CHUNK
    mesh = plsc.VectorSubcoreMesh(core_axis_name="c", subcore_axis_name="s")

    @functools.partial(
        pl.kernel,
        mesh=mesh,
        out_type=jax.ShapeDtypeStruct((B, D), jnp.float32),
        scratch_types=[
            pltpu.VMEM((n_sub_rows, SUB), jnp.int32),
            pltpu.VMEM((CHUNK, D), jnp.float32),
            pltpu.SemaphoreType.DMA,
        ],
    )
    def body(idx_hbm, table_hbm, out_hbm, idx_v, rows_v, gsem):
        wid = lax.axis_index("s") * NC + lax.axis_index("c")
        base = wid * b_per_w
        pltpu.sync_copy(idx_hbm.at[wid], idx_v)

        def chunk_body(c, carry):
            copies = []
            for b in range(SUBS_PER_CHUNK):
                copies.append(pltpu.async_copy(
                    table_hbm.at[idx_v.at[c * SUBS_PER_CHUNK + b]],
                    rows_v.at[pl.ds(b * SUB, SUB)],
                    gsem,
                ))
            for cp in copies:
                cp.wait()
            pltpu.sync_copy(rows_v, out_hbm.at[pl.ds(base + c * CHUNK, CHUNK)])
            return carry

        lax.fori_loop(0, n_chunks, chunk_body, 0)

    return body


def kernel(X, table):
    Bt, S, _ = X.shape
    B = Bt * S
    idx = X[:, :, VAR_IDX].reshape(NW, B // (NW * SUB), SUB)
    out = _make_gather(B)(idx, table)
    return out.reshape(Bt, S, D)


# speedup vs baseline: 1.4757x; 1.4757x over previous
import functools

import jax
import jax.numpy as jnp
from jax import lax
from jax.experimental import pallas as pl
from jax.experimental.pallas import tpu as pltpu
from jax.experimental.pallas import tpu_sc as plsc

VAR_IDX = 3
D = 32
NC = 2
NS = 16
NW = NC * NS
SUB = 128
SUBS_PER_CHUNK = 8
CHUNK = SUB * SUBS_PER_CHUNK


def _make_gather(B):
    b_per_w = B // NW
    n_sub_rows = b_per_w // SUB
    n_chunks = b_per_w // CHUNK
    mesh = plsc.VectorSubcoreMesh(core_axis_name="c", subcore_axis_name="s")

    @functools.partial(
        pl.kernel,
        mesh=mesh,
        out_type=jax.ShapeDtypeStruct((B, D), jnp.float32),
        scratch_types=[
            pltpu.VMEM((n_sub_rows, SUB), jnp.int32),
            pltpu.VMEM((CHUNK, D), jnp.float32),
            pltpu.SemaphoreType.DMA,
        ],
        compiler_params=pltpu.CompilerParams(use_tc_tiling_on_sc=False),
    )
    def body(idx_hbm, table_hbm, out_hbm, idx_v, rows_v, gsem):
        wid = lax.axis_index("s") * NC + lax.axis_index("c")
        base = wid * b_per_w
        pltpu.sync_copy(idx_hbm.at[wid], idx_v)

        def chunk_body(c, carry):
            copies = []
            for b in range(SUBS_PER_CHUNK):
                copies.append(pltpu.async_copy(
                    table_hbm.at[idx_v.at[c * SUBS_PER_CHUNK + b]],
                    rows_v.at[pl.ds(b * SUB, SUB)],
                    gsem,
                ))
            for cp in copies:
                cp.wait()
            pltpu.sync_copy(rows_v, out_hbm.at[pl.ds(base + c * CHUNK, CHUNK)])
            return carry

        lax.fori_loop(0, n_chunks, chunk_body, 0)

    return body


def kernel(X, table):
    Bt, S, _ = X.shape
    B = Bt * S
    idx = X[:, :, VAR_IDX].reshape(NW, B // (NW * SUB), SUB)
    out = _make_gather(B)(idx, table)
    return out.reshape(Bt, S, D)


# speedup vs baseline: 1.4988x; 1.0157x over previous
import functools

import jax
import jax.numpy as jnp
from jax import lax
from jax.experimental import pallas as pl
from jax.experimental.pallas import tpu as pltpu
from jax.experimental.pallas import tpu_sc as plsc

VAR_IDX = 3
D = 32
NC = 2
NS = 16
NW = NC * NS
SUB = 128
SUBS_PER_CHUNK = 10
CHUNK = SUB * SUBS_PER_CHUNK


def _make_gather(B):
    b_per_w = B // NW
    n_sub_rows = b_per_w // SUB
    n_chunks = b_per_w // CHUNK
    n_pairs = n_chunks // 2
    mesh = plsc.VectorSubcoreMesh(core_axis_name="c", subcore_axis_name="s")

    @functools.partial(
        pl.kernel,
        mesh=mesh,
        out_type=jax.ShapeDtypeStruct((B, D), jnp.float32),
        scratch_types=[
            pltpu.VMEM((n_sub_rows, SUB), jnp.int32),
            pltpu.VMEM((CHUNK, D), jnp.float32),
            pltpu.VMEM((CHUNK, D), jnp.float32),
            pltpu.SemaphoreType.DMA,
            pltpu.SemaphoreType.DMA,
            pltpu.SemaphoreType.DMA,
            pltpu.SemaphoreType.DMA,
        ],
        compiler_params=pltpu.CompilerParams(use_tc_tiling_on_sc=False),
    )
    def body(idx_hbm, table_hbm, out_hbm, idx_v, rows0, rows1,
             gsem0, gsem1, osem0, osem1):
        wid = lax.axis_index("s") * NC + lax.axis_index("c")
        base = wid * b_per_w
        pltpu.sync_copy(idx_hbm.at[wid], idx_v)

        def fire_g(c, rows, gsem):
            for k in range(SUBS_PER_CHUNK):
                pltpu.async_copy(
                    table_hbm.at[idx_v.at[c * SUBS_PER_CHUNK + k]],
                    rows.at[pl.ds(k * SUB, SUB)],
                    gsem,
                )

        def wait_g(rows, gsem):
            pltpu.make_async_copy(
                out_hbm.at[pl.ds(base, CHUNK)], rows, gsem).wait()

        def fire_w(c, rows, osem):
            pltpu.async_copy(
                rows, out_hbm.at[pl.ds(base + c * CHUNK, CHUNK)], osem)

        def wait_w(rows, osem):
            pltpu.make_async_copy(
                rows, out_hbm.at[pl.ds(base, CHUNK)], osem).wait()

        fire_g(0, rows0, gsem0)

        def pair(j, carry):
            @pl.when(j > 0)
            def _():
                wait_w(rows1, osem1)
            fire_g(2 * j + 1, rows1, gsem1)
            wait_g(rows0, gsem0)
            fire_w(2 * j, rows0, osem0)
            wait_w(rows0, osem0)

            @pl.when(j < n_pairs - 1)
            def _():
                fire_g(2 * j + 2, rows0, gsem0)
            wait_g(rows1, gsem1)
            fire_w(2 * j + 1, rows1, osem1)
            return carry

        lax.fori_loop(0, n_pairs, pair, 0)
        wait_w(rows1, osem1)

    return body


def kernel(X, table):
    Bt, S, _ = X.shape
    B = Bt * S
    idx = X[:, :, VAR_IDX].reshape(NW, B // (NW * SUB), SUB)
    out = _make_gather(B)(idx, table)
    return out.reshape(Bt, S, D)
